# baseline (speedup 1.0000x reference)
import jax
import jax.numpy as jnp
from jax import lax
from jax.experimental import pallas as pl
from jax.experimental.pallas import tpu as pltpu
from jax.experimental.pallas import tpu_sc as plsc

L = 16
NC = 2
NS = 16
NW = NC * NS
B = 16384
D = 32
BPW = B // NW
IDXW = 128
NCHUNK = BPW // IDXW
GROUPS = BPW // L
BFOLD = 16


def _mirt_body(user_hbm, item_hbm, idiv_hbm, imod_hbm,
               theta_hbm, a_hbm, b16_hbm, out_hbm,
               uidx_v, iidx_v, idiv_v, imod_v,
               theta_v, a_v, brows_v, out_v,
               sem_t, sem_a, sem_b):
    c = lax.axis_index("c")
    s = lax.axis_index("s")
    wid = s * NC + c

    rowbase = wid * NCHUNK
    pltpu.sync_copy(user_hbm.at[pl.ds(rowbase, NCHUNK)], uidx_v)
    pltpu.sync_copy(item_hbm.at[pl.ds(rowbase, NCHUNK)], iidx_v)
    pltpu.sync_copy(idiv_hbm.at[pl.ds(rowbase, NCHUNK)], idiv_v)
    pltpu.sync_copy(imod_hbm.at[pl.ds(wid * BPW, BPW)], imod_v)

    copies = []
    for j in range(NCHUNK):
        copies.append(pltpu.async_copy(
            theta_hbm.at[uidx_v.at[j]], theta_v.at[pl.ds(j * IDXW, IDXW)],
            sem_t))
        copies.append(pltpu.async_copy(
            a_hbm.at[iidx_v.at[j]], a_v.at[pl.ds(j * IDXW, IDXW)],
            sem_a))
        copies.append(pltpu.async_copy(
            b16_hbm.at[idiv_v.at[j]], brows_v.at[pl.ds(j * IDXW, IDXW)],
            sem_b))
    for cp in copies:
        cp.wait()

    lanes = lax.iota(jnp.int32, L)

    def group(g, carry):
        base = pl.multiple_of(g * L, L)
        rows = g * L + lanes
        acc = jnp.zeros((L,), jnp.float32)
        for d in range(D):
            dcol = jnp.full((L,), d, jnp.int32)
            tv = plsc.load_gather(theta_v, [rows, dcol])
            av = plsc.load_gather(a_v, [rows, dcol])
            acc = acc + tv * av
        bcol = imod_v[pl.ds(base, L)]
        bv = plsc.load_gather(brows_v, [rows, bcol])
        res = 1.0 / (1.0 + jnp.exp(-acc + bv))
        out_v[pl.ds(base, L)] = res
        return carry

    lax.fori_loop(0, GROUPS, group, 0)

    pltpu.sync_copy(out_v, out_hbm.at[pl.ds(wid * BPW, BPW)])


@jax.jit
def _mirt(user2d, item2d, idiv2d, imod, theta_table, a_table, b16):
    mesh = plsc.VectorSubcoreMesh(core_axis_name="c", subcore_axis_name="s")
    fn = pl.kernel(
        _mirt_body,
        mesh=mesh,
        compiler_params=pltpu.CompilerParams(needs_layout_passes=False,
                                             use_tc_tiling_on_sc=False),
        out_type=jax.ShapeDtypeStruct((B,), jnp.float32),
        scratch_types=[
            pltpu.VMEM((NCHUNK, IDXW), jnp.int32),
            pltpu.VMEM((NCHUNK, IDXW), jnp.int32),
            pltpu.VMEM((NCHUNK, IDXW), jnp.int32),
            pltpu.VMEM((BPW,), jnp.int32),
            pltpu.VMEM((BPW, D), jnp.float32),
            pltpu.VMEM((BPW, D), jnp.float32),
            pltpu.VMEM((BPW, BFOLD), jnp.float32),
            pltpu.VMEM((BPW,), jnp.float32),
            pltpu.SemaphoreType.DMA,
            pltpu.SemaphoreType.DMA,
            pltpu.SemaphoreType.DMA,
        ],
    )
    return fn(user2d, item2d, idiv2d, imod, theta_table, a_table, b16)


def kernel(user, item, theta_table, a_table, b_table):
    user = user.astype(jnp.int32)
    item = item.astype(jnp.int32)
    user2d = user.reshape(B // IDXW, IDXW)
    item2d = item.reshape(B // IDXW, IDXW)
    idiv2d = (item // BFOLD).reshape(B // IDXW, IDXW)
    imod = item % BFOLD
    b16 = b_table.reshape(b_table.shape[0] // BFOLD, BFOLD)
    return _mirt(user2d, item2d, idiv2d, imod, theta_table, a_table, b16)


# speedup vs baseline: 1.3108x; 1.3108x over previous
import jax
import jax.numpy as jnp
from jax import lax
from jax.experimental import pallas as pl
from jax.experimental.pallas import tpu as pltpu
from jax.experimental.pallas import tpu_sc as plsc

L = 16
NC = 2
NS = 16
NW = NC * NS
B = 16384
D = 32
BPW = B // NW
G = BPW // L
BFOLD = 16
BROWS = 6256


def _issue_group(gb, uidx_v, iidx_v, theta_hbm, a_hbm, b16_hbm,
                 tbuf, abuf, bbuf, sem):
    uvec = uidx_v[pl.ds(gb, L)]
    ivec = iidx_v[pl.ds(gb, L)]
    for e in range(L):
        r = uvec[e]
        q8 = pl.multiple_of((r // 8) * 8, 8)
        pltpu.async_copy(theta_hbm.at[pl.ds(q8, 8), :],
                         tbuf.at[pl.ds(e * 8, 8), :], sem)
    for e in range(L):
        r = ivec[e]
        q8 = pl.multiple_of((r // 8) * 8, 8)
        pltpu.async_copy(a_hbm.at[pl.ds(q8, 8), :],
                         abuf.at[pl.ds(e * 8, 8), :], sem)
    for e in range(L):
        rb = ivec[e] // BFOLD
        q8 = pl.multiple_of((rb // 8) * 8, 8)
        pltpu.async_copy(b16_hbm.at[pl.ds(q8, 8), :],
                         bbuf.at[pl.ds(e * 8, 8), :], sem)


def _drain_group(theta_hbm, a_hbm, b16_hbm, tbuf, abuf, bbuf, sem):
    pltpu.make_async_copy(theta_hbm.at[pl.ds(0, L * 8), :], tbuf, sem).wait()
    pltpu.make_async_copy(a_hbm.at[pl.ds(0, L * 8), :], abuf, sem).wait()
    pltpu.make_async_copy(b16_hbm.at[pl.ds(0, L * 8), :], bbuf, sem).wait()


def _mirt_body(user_hbm, item_hbm, um8_hbm, im8_hbm, ib8_hbm, ic_hbm,
               theta_hbm, a_hbm, b16_hbm, out_hbm,
               uidx_v, iidx_v, um8_v, im8_v, ib8_v, ic_v,
               tbuf0, abuf0, bbuf0, tbuf1, abuf1, bbuf1, out_v,
               sem0, sem1):
    c = lax.axis_index("c")
    s = lax.axis_index("s")
    wid = s * NC + c
    base = wid * BPW

    for src, dst in ((user_hbm, uidx_v), (item_hbm, iidx_v),
                     (um8_hbm, um8_v), (im8_hbm, im8_v),
                     (ib8_hbm, ib8_v), (ic_hbm, ic_v)):
        pltpu.sync_copy(src.at[pl.ds(base, BPW)], dst)

    lanes = lax.iota(jnp.int32, L)
    lanes8 = lanes * 8

    def compute(gb, tbuf, abuf, bbuf):
        trows = lanes8 + um8_v[pl.ds(gb, L)]
        arows = lanes8 + im8_v[pl.ds(gb, L)]
        acc = jnp.zeros((L,), jnp.float32)
        for d in range(D):
            dcol = jnp.full((L,), d, jnp.int32)
            tv = plsc.load_gather(tbuf, [trows, dcol])
            av = plsc.load_gather(abuf, [arows, dcol])
            acc = acc + tv * av
        brows = lanes8 + ib8_v[pl.ds(gb, L)]
        bv = plsc.load_gather(bbuf, [brows, ic_v[pl.ds(gb, L)]])
        out_v[pl.ds(gb, L)] = 1.0 / (1.0 + jnp.exp(-acc + bv))

    bufs = ((tbuf0, abuf0, bbuf0), (tbuf1, abuf1, bbuf1))
    sems = (sem0, sem1)

    def issue(g, bi):
        _issue_group(pl.multiple_of(g * L, L), uidx_v, iidx_v,
                     theta_hbm, a_hbm, b16_hbm, *bufs[bi], sems[bi])

    def drain(bi):
        _drain_group(theta_hbm, a_hbm, b16_hbm, *bufs[bi], sems[bi])

    issue(0, 0)

    def pair(k, carry):
        g0 = k * 2
        issue(g0 + 1, 1)
        drain(0)
        compute(pl.multiple_of(g0 * L, L), *bufs[0])

        @pl.when(k < G // 2 - 1)
        def _():
            issue(g0 + 2, 0)

        drain(1)
        compute(pl.multiple_of((g0 + 1) * L, L), *bufs[1])
        return carry

    lax.fori_loop(0, G // 2, pair, 0)

    pltpu.sync_copy(out_v, out_hbm.at[pl.ds(base, BPW)])


@jax.jit
def _mirt(user, item, um8, im8, ib8, ic, theta_table, a_table, b16):
    mesh = plsc.VectorSubcoreMesh(core_axis_name="c", subcore_axis_name="s")
    fn = pl.kernel(
        _mirt_body,
        mesh=mesh,
        compiler_params=pltpu.CompilerParams(needs_layout_passes=False,
                                             use_tc_tiling_on_sc=True),
        out_type=jax.ShapeDtypeStruct((B,), jnp.float32),
        scratch_types=[
            pltpu.VMEM((BPW,), jnp.int32),
            pltpu.VMEM((BPW,), jnp.int32),
            pltpu.VMEM((BPW,), jnp.int32),
            pltpu.VMEM((BPW,), jnp.int32),
            pltpu.VMEM((BPW,), jnp.int32),
            pltpu.VMEM((BPW,), jnp.int32),
            pltpu.VMEM((L * 8, D), jnp.float32),
            pltpu.VMEM((L * 8, D), jnp.float32),
            pltpu.VMEM((L * 8, BFOLD), jnp.float32),
            pltpu.VMEM((L * 8, D), jnp.float32),
            pltpu.VMEM((L * 8, D), jnp.float32),
            pltpu.VMEM((L * 8, BFOLD), jnp.float32),
            pltpu.VMEM((BPW,), jnp.float32),
            pltpu.SemaphoreType.DMA,
            pltpu.SemaphoreType.DMA,
        ],
    )
    return fn(user, item, um8, im8, ib8, ic, theta_table, a_table, b16)


def kernel(user, item, theta_table, a_table, b_table):
    user = user.astype(jnp.int32)
    item = item.astype(jnp.int32)
    um8 = user % 8
    im8 = item % 8
    ib8 = (item // BFOLD) % 8
    ic = item % BFOLD
    b16 = jnp.pad(b_table.reshape(b_table.shape[0] // BFOLD, BFOLD),
                  ((0, BROWS - b_table.shape[0] // BFOLD), (0, 0)))
    return _mirt(user, item, um8, im8, ib8, ic, theta_table, a_table, b16)


# speedup vs baseline: 2.5739x; 1.9636x over previous
import jax
import jax.numpy as jnp
from jax import lax
from jax.experimental import pallas as pl
from jax.experimental.pallas import tpu as pltpu
from jax.experimental.pallas import tpu_sc as plsc

L = 16
NC = 2
NS = 16
NW = NC * NS
B = 16384
D = 32
BPW = B // NW
G = BPW // L
BFOLD = 16
BROWS = 6256
TCOL = 128


def _mirt_body(user_hbm, item_hbm, uc_hbm, im8_hbm, ib8_hbm, ic_hbm,
               theta_t_hbm, a_hbm, b16_hbm, out_hbm,
               uidx_v, iidx_v, uc_v, im8_v, ib8_v, ic_v,
               tbuf, abuf, bbuf, out_v, sem):
    c = lax.axis_index("c")
    s = lax.axis_index("s")
    wid = s * NC + c
    base = wid * BPW

    for src, dst in ((user_hbm, uidx_v), (item_hbm, iidx_v),
                     (uc_hbm, uc_v), (im8_hbm, im8_v),
                     (ib8_hbm, ib8_v), (ic_hbm, ic_v)):
        pltpu.sync_copy(src.at[pl.ds(base, BPW)], dst)

    lanes = lax.iota(jnp.int32, L)
    lanes8 = lanes * 8
    lanes32 = lanes * D

    def group(g, carry):
        gb = pl.multiple_of(g * L, L)
        uvec = uidx_v[pl.ds(gb, L)]
        ivec = iidx_v[pl.ds(gb, L)]
        for e in range(L):
            r = uvec[e]
            c128 = pl.multiple_of((r // TCOL) * TCOL, TCOL)
            pltpu.async_copy(theta_t_hbm.at[:, pl.ds(c128, TCOL)],
                             tbuf.at[pl.ds(e * D, D), :], sem)
        for e in range(L):
            r = ivec[e]
            q8 = pl.multiple_of((r // 8) * 8, 8)
            pltpu.async_copy(a_hbm.at[pl.ds(q8, 8), :],
                             abuf.at[pl.ds(e * 8, 8), :], sem)
        for e in range(L):
            rb = ivec[e] // BFOLD
            q8 = pl.multiple_of((rb // 8) * 8, 8)
            pltpu.async_copy(b16_hbm.at[pl.ds(q8, 8), :],
                             bbuf.at[pl.ds(e * 8, 8), :], sem)
        for e in range(L):
            pltpu.make_async_copy(theta_t_hbm.at[:, pl.ds(0, TCOL)],
                                  tbuf.at[pl.ds(e * D, D), :], sem).wait()
        pltpu.make_async_copy(a_hbm.at[pl.ds(0, L * 8), :], abuf, sem).wait()
        pltpu.make_async_copy(b16_hbm.at[pl.ds(0, L * 8), :], bbuf, sem).wait()

        tcols = uc_v[pl.ds(gb, L)]
        arows = lanes8 + im8_v[pl.ds(gb, L)]
        acc = jnp.zeros((L,), jnp.float32)
        for d in range(D):
            dcol = jnp.full((L,), d, jnp.int32)
            tv = plsc.load_gather(tbuf, [lanes32 + d, tcols])
            av = plsc.load_gather(abuf, [arows, dcol])
            acc = acc + tv * av
        brows = lanes8 + ib8_v[pl.ds(gb, L)]
        bv = plsc.load_gather(bbuf, [brows, ic_v[pl.ds(gb, L)]])
        out_v[pl.ds(gb, L)] = 1.0 / (1.0 + jnp.exp(-acc + bv))
        return carry

    lax.fori_loop(0, G, group, 0)

    pltpu.sync_copy(out_v, out_hbm.at[pl.ds(base, BPW)])


@jax.jit
def _mirt(user, item, uc, im8, ib8, ic, theta_t, a_table, b16):
    mesh = plsc.VectorSubcoreMesh(core_axis_name="c", subcore_axis_name="s")
    fn = pl.kernel(
        _mirt_body,
        mesh=mesh,
        compiler_params=pltpu.CompilerParams(needs_layout_passes=False,
                                             use_tc_tiling_on_sc=True),
        out_type=jax.ShapeDtypeStruct((B,), jnp.float32),
        scratch_types=[
            pltpu.VMEM((BPW,), jnp.int32),
            pltpu.VMEM((BPW,), jnp.int32),
            pltpu.VMEM((BPW,), jnp.int32),
            pltpu.VMEM((BPW,), jnp.int32),
            pltpu.VMEM((BPW,), jnp.int32),
            pltpu.VMEM((BPW,), jnp.int32),
            pltpu.VMEM((L * D, TCOL), jnp.float32),
            pltpu.VMEM((L * 8, D), jnp.float32),
            pltpu.VMEM((L * 8, BFOLD), jnp.float32),
            pltpu.VMEM((BPW,), jnp.float32),
            pltpu.SemaphoreType.DMA,
        ],
    )
    return fn(user, item, uc, im8, ib8, ic, theta_t, a_table, b16)


def kernel(user, item, theta_table, a_table, b_table):
    user = user.astype(jnp.int32)
    item = item.astype(jnp.int32)
    uc = user % TCOL
    im8 = item % 8
    ib8 = (item // BFOLD) % 8
    ic = item % BFOLD
    b16 = jnp.pad(b_table.reshape(b_table.shape[0] // BFOLD, BFOLD),
                  ((0, BROWS - b_table.shape[0] // BFOLD), (0, 0)))
    return _mirt(user, item, uc, im8, ib8, ic, theta_table.T, a_table, b16)
